# baseline (speedup 1.0000x reference)
import functools

import jax
import jax.numpy as jnp
from jax import lax
from jax.experimental import pallas as pl
from jax.experimental.pallas import tpu as pltpu
from jax.experimental.pallas import tpu_sc as plsc

_NC = 2
_NS = 16
_K = 40
_SLOPE = 0.01


def _lrelu(x):
    return jnp.where(x >= 0, x, _SLOPE * x)


def _tc_init(feature, semantic, w1s, b1s2, w1m, b1m2, alpha2, beta2, blk):
    n, f = feature.shape
    s_dim = semantic.shape[1]
    h = w1s.shape[0]

    def body(f_r, se_r, w1s_r, b1s_r, w1m_r, b1m_r, a_r, b_r, o_r):
        a = a_r[0, 0]
        b = b_r[0, 0]
        x = lax.dot_general(f_r[...], w1s_r[...], (((1,), (1,)), ((), ())),
                            preferred_element_type=jnp.float32) + b1s_r[...]
        y = lax.dot_general(se_r[...], w1m_r[...], (((1,), (1,)), ((), ())),
                            preferred_element_type=jnp.float32) + b1m_r[...]
        o_r[...] = a * _lrelu(x) + b * _lrelu(y)

    return pl.pallas_call(
        body,
        grid=(n // blk,),
        in_specs=[
            pl.BlockSpec((blk, f), lambda i: (i, 0)),
            pl.BlockSpec((blk, s_dim), lambda i: (i, 0)),
            pl.BlockSpec((h, f), lambda i: (0, 0)),
            pl.BlockSpec((1, h), lambda i: (0, 0)),
            pl.BlockSpec((h, s_dim), lambda i: (0, 0)),
            pl.BlockSpec((1, h), lambda i: (0, 0)),
            pl.BlockSpec((1, 1), lambda i: (0, 0)),
            pl.BlockSpec((1, 1), lambda i: (0, 0)),
        ],
        out_specs=pl.BlockSpec((blk, h), lambda i: (i, 0)),
        out_shape=jax.ShapeDtypeStruct((n, h), jnp.float32),
    )(feature, semantic, w1s, b1s2, w1m, b1m2, alpha2, beta2)


def _tc_dinv(degparts, npad, h):
    def body(d_r, o_r):
        dv = d_r[...]
        o_r[...] = 1.0 / jnp.maximum(dv[0] + dv[1], 1.0)

    return pl.pallas_call(
        body,
        out_shape=jax.ShapeDtypeStruct((npad, h), jnp.float32),
    )(degparts)


def _tc_update(emb, parts, dinv, w2, b22, alpha2, beta2, blk):
    n, h = emb.shape

    def body(e_r, p_r, d_r, w2_r, b2_r, a_r, b_r, o_r):
        a = a_r[0, 0]
        b = b_r[0, 0]
        pv = p_r[...]
        msum = pv[0] + pv[1]
        comb = a * e_r[...] + b * (msum * d_r[...])
        z = lax.dot_general(comb, w2_r[...], (((1,), (1,)), ((), ())),
                            preferred_element_type=jnp.float32) + b2_r[...]
        o_r[...] = _lrelu(z)

    return pl.pallas_call(
        body,
        grid=(n // blk,),
        in_specs=[
            pl.BlockSpec((blk, h), lambda i: (i, 0)),
            pl.BlockSpec((_NC, blk, h), lambda i: (0, i, 0)),
            pl.BlockSpec((blk, h), lambda i: (i, 0)),
            pl.BlockSpec((h, h), lambda i: (0, 0)),
            pl.BlockSpec((1, h), lambda i: (0, 0)),
            pl.BlockSpec((1, 1), lambda i: (0, 0)),
            pl.BlockSpec((1, 1), lambda i: (0, 0)),
        ],
        out_specs=pl.BlockSpec((blk, h), lambda i: (i, 0)),
        out_shape=jax.ShapeDtypeStruct((n, h), jnp.float32),
    )(emb, parts, dinv, w2, b22, alpha2, beta2)


def _sc_degree(dst3, ones_rows, zrows, npad, h, nchunk):
    nps = npad // _NS
    win = 8
    mesh = plsc.VectorSubcoreMesh(core_axis_name="c", subcore_axis_name="s",
                                  num_cores=_NC, num_subcores=_NS)

    @functools.partial(
        pl.kernel,
        out_type=jax.ShapeDtypeStruct((_NC, npad, h), jnp.float32),
        mesh=mesh,
        scratch_types=[
            pltpu.VMEM((nchunk, _K), jnp.int32),
            pltpu.VMEM((_K, h), jnp.float32),
            pltpu.VMEM_SHARED((npad, h), jnp.float32),
            pltpu.SemaphoreType.DMA,
        ],
    )
    def deg_kernel(dst3_h, ones_h, z_h, out_h, di2, ones_v, acc, ssem):
        c = lax.axis_index("c")
        s = lax.axis_index("s")
        wid = c * _NS + s
        row0 = s * nps
        pltpu.sync_copy(ones_h, ones_v)
        pltpu.sync_copy(dst3_h.at[wid], di2)
        pltpu.sync_copy(z_h, acc.at[pl.ds(row0, nps)])
        plsc.subcore_barrier()

        def s_start(j):
            pltpu.async_copy(ones_v, acc.at[di2.at[j]], ssem, add=True)

        def s_wait():
            pltpu.make_async_copy(ones_v, acc.at[di2.at[0]], ssem).wait()

        for b in range(win):
            s_start(b)

        def body(j, carry):
            s_wait()
            s_start(j + win)
            return carry

        lax.fori_loop(0, nchunk - win, body, 0)
        for _ in range(win):
            s_wait()
        plsc.subcore_barrier()
        pltpu.sync_copy(acc.at[pl.ds(row0, nps)], out_h.at[c, pl.ds(row0, nps)])

    return deg_kernel(dst3, ones_rows, zrows)


_NB = 8


def _sc_segsum(emb, src4, dst4, zrows, npad, h, ngroups):
    nps = npad // _NS
    assert ngroups % 2 == 0 and ngroups >= 4
    mesh = plsc.VectorSubcoreMesh(core_axis_name="c", subcore_axis_name="s",
                                  num_cores=_NC, num_subcores=_NS)

    @functools.partial(
        pl.kernel,
        out_type=jax.ShapeDtypeStruct((_NC, npad, h), jnp.float32),
        mesh=mesh,
        scratch_types=[
            [pltpu.VMEM((_NB, _K), jnp.int32)] * 2,
            [pltpu.VMEM((_NB, _K), jnp.int32)] * 2,
            [pltpu.VMEM((_K, h), jnp.float32)] * _NB,
            pltpu.VMEM_SHARED((npad, h), jnp.float32),
            [pltpu.SemaphoreType.DMA] * 2,
            [pltpu.SemaphoreType.DMA] * _NB,
            [pltpu.SemaphoreType.DMA] * _NB,
        ],
    )
    def seg_kernel(emb_h, src4_h, dst4_h, z_h, out_h, is2, id2, rows, acc,
                   isem, gsem, ssem):
        c = lax.axis_index("c")
        s = lax.axis_index("s")
        wid = c * _NS + s
        row0 = s * nps

        def i_start(g, p):
            pltpu.async_copy(src4_h.at[wid, g], is2[p], isem[p])
            pltpu.async_copy(dst4_h.at[wid, g], id2[p], isem[p])

        def i_wait(p):
            pltpu.make_async_copy(src4_h.at[wid, 0], is2[p], isem[p]).wait()
            pltpu.make_async_copy(src4_h.at[wid, 0], id2[p], isem[p]).wait()

        def g_start(p, b):
            pltpu.async_copy(emb_h.at[is2[p].at[b]], rows[b], gsem[b])

        def g_wait(p, b):
            pltpu.make_async_copy(emb_h.at[is2[p].at[b]], rows[b],
                                  gsem[b]).wait()

        def s_start(p, b):
            pltpu.async_copy(rows[b], acc.at[id2[p].at[b]], ssem[b], add=True)

        def s_wait(p, b):
            pltpu.make_async_copy(rows[b], acc.at[id2[p].at[b]],
                                  ssem[b]).wait()

        def process_group(g, p, pre):
            if pre:
                i_start(g + 1, 1 - p)
            for b in range(_NB):
                g_wait(p, b)
                s_start(p, b)
            if pre:
                i_wait(1 - p)
            for b in range(_NB):
                s_wait(p, b)
                if pre:
                    g_start(1 - p, b)

        i_start(0, 0)
        pltpu.sync_copy(z_h, acc.at[pl.ds(row0, nps)])
        plsc.subcore_barrier()
        i_wait(0)
        for b in range(_NB):
            g_start(0, b)

        def pair(t, carry):
            process_group(2 * t, 0, True)
            process_group(2 * t + 1, 1, True)
            return carry

        lax.fori_loop(0, ngroups // 2 - 1, pair, 0)
        process_group(ngroups - 2, 0, True)
        process_group(ngroups - 1, 1, False)

        plsc.subcore_barrier()
        pltpu.sync_copy(acc.at[pl.ds(row0, nps)], out_h.at[c, pl.ds(row0, nps)])

    return seg_kernel(emb, src4, dst4, zrows)


def kernel(feature, semantic, edge_index, W1s, b1s, W1m, b1m, W2, b2, alpha,
           beta, num_iterations):
    n, _ = feature.shape
    h = W2.shape[0]
    e = edge_index.shape[1]
    nw = _NC * _NS

    npad = -(-(n + 1) // (_NS * 8)) * (_NS * 8)

    gsz = _NB * _K
    ngroups = -(-e // (nw * gsz))
    if ngroups % 2:
        ngroups += 1
    e_pad = nw * ngroups * gsz
    src_p = jnp.concatenate(
        [edge_index[0], jnp.zeros((e_pad - e,), jnp.int32)])
    dst_p = jnp.concatenate(
        [edge_index[1], jnp.full((e_pad - e,), n, jnp.int32)])
    nchunk = ngroups * _NB
    src4 = src_p.reshape(nw, ngroups, _NB, _K)
    dst4 = dst_p.reshape(nw, ngroups, _NB, _K)
    dst3 = dst_p.reshape(nw, nchunk, _K)

    alpha2 = jnp.asarray(alpha, jnp.float32).reshape(1, 1)
    beta2 = jnp.asarray(beta, jnp.float32).reshape(1, 1)
    b1s2 = b1s.reshape(1, -1)
    b1m2 = b1m.reshape(1, -1)
    b22 = b2.reshape(1, -1)

    nps = npad // _NS
    zrows = jnp.zeros((nps, h), jnp.float32)
    ones_rows = jnp.ones((_K, h), jnp.float32)

    degp = _sc_degree(dst3, ones_rows, zrows, npad, h, nchunk)
    dinv = _tc_dinv(degp, npad, h)
    emb0 = _tc_init(feature, semantic, W1s, b1s2, W1m, b1m2, alpha2, beta2,
                    blk=2000)

    def body(_, emb):
        parts = _sc_segsum(emb, src4, dst4, zrows, npad, h, ngroups)
        return _tc_update(emb, parts, dinv, W2, b22, alpha2, beta2, blk=2000)

    return lax.fori_loop(0, num_iterations, body, emb0)


# speedup vs baseline: 1.1709x; 1.1709x over previous
import functools

import jax
import jax.numpy as jnp
from jax import lax
from jax.experimental import pallas as pl
from jax.experimental.pallas import tpu as pltpu
from jax.experimental.pallas import tpu_sc as plsc

_NC = 2
_NS = 16
_K = 80
_SLOPE = 0.01


def _lrelu(x):
    return jnp.where(x >= 0, x, _SLOPE * x)


def _tc_init(feature, semantic, w1s, b1s2, w1m, b1m2, alpha2, beta2, blk):
    n, f = feature.shape
    s_dim = semantic.shape[1]
    h = w1s.shape[0]

    def body(f_r, se_r, w1s_r, b1s_r, w1m_r, b1m_r, a_r, b_r, o_r):
        a = a_r[0, 0]
        b = b_r[0, 0]
        x = lax.dot_general(f_r[...], w1s_r[...], (((1,), (1,)), ((), ())),
                            preferred_element_type=jnp.float32) + b1s_r[...]
        y = lax.dot_general(se_r[...], w1m_r[...], (((1,), (1,)), ((), ())),
                            preferred_element_type=jnp.float32) + b1m_r[...]
        o_r[...] = a * _lrelu(x) + b * _lrelu(y)

    return pl.pallas_call(
        body,
        grid=(n // blk,),
        in_specs=[
            pl.BlockSpec((blk, f), lambda i: (i, 0)),
            pl.BlockSpec((blk, s_dim), lambda i: (i, 0)),
            pl.BlockSpec((h, f), lambda i: (0, 0)),
            pl.BlockSpec((1, h), lambda i: (0, 0)),
            pl.BlockSpec((h, s_dim), lambda i: (0, 0)),
            pl.BlockSpec((1, h), lambda i: (0, 0)),
            pl.BlockSpec((1, 1), lambda i: (0, 0)),
            pl.BlockSpec((1, 1), lambda i: (0, 0)),
        ],
        out_specs=pl.BlockSpec((blk, h), lambda i: (i, 0)),
        out_shape=jax.ShapeDtypeStruct((n, h), jnp.float32),
    )(feature, semantic, w1s, b1s2, w1m, b1m2, alpha2, beta2)


def _tc_dinv(degparts, npad, h):
    def body(d_r, o_r):
        dv = d_r[...]
        o_r[...] = 1.0 / jnp.maximum(dv[0] + dv[1], 1.0)

    return pl.pallas_call(
        body,
        out_shape=jax.ShapeDtypeStruct((npad, h), jnp.float32),
    )(degparts)


def _tc_update(emb, parts, dinv, w2, b22, alpha2, beta2, blk):
    n, h = emb.shape

    def body(e_r, p_r, d_r, w2_r, b2_r, a_r, b_r, o_r):
        a = a_r[0, 0]
        b = b_r[0, 0]
        pv = p_r[...]
        msum = pv[0] + pv[1]
        comb = a * e_r[...] + b * (msum * d_r[...])
        z = lax.dot_general(comb, w2_r[...], (((1,), (1,)), ((), ())),
                            preferred_element_type=jnp.float32) + b2_r[...]
        o_r[...] = _lrelu(z)

    return pl.pallas_call(
        body,
        grid=(n // blk,),
        in_specs=[
            pl.BlockSpec((blk, h), lambda i: (i, 0)),
            pl.BlockSpec((_NC, blk, h), lambda i: (0, i, 0)),
            pl.BlockSpec((blk, h), lambda i: (i, 0)),
            pl.BlockSpec((h, h), lambda i: (0, 0)),
            pl.BlockSpec((1, h), lambda i: (0, 0)),
            pl.BlockSpec((1, 1), lambda i: (0, 0)),
            pl.BlockSpec((1, 1), lambda i: (0, 0)),
        ],
        out_specs=pl.BlockSpec((blk, h), lambda i: (i, 0)),
        out_shape=jax.ShapeDtypeStruct((n, h), jnp.float32),
    )(emb, parts, dinv, w2, b22, alpha2, beta2)


def _sc_degree(dst3, ones_rows, zrows, npad, h, nchunk):
    nps = npad // _NS
    win = 8
    mesh = plsc.VectorSubcoreMesh(core_axis_name="c", subcore_axis_name="s",
                                  num_cores=_NC, num_subcores=_NS)

    @functools.partial(
        pl.kernel,
        out_type=jax.ShapeDtypeStruct((_NC, npad, h), jnp.float32),
        mesh=mesh,
        scratch_types=[
            pltpu.VMEM((nchunk, _K), jnp.int32),
            pltpu.VMEM((_K, h), jnp.float32),
            pltpu.VMEM_SHARED((npad, h), jnp.float32),
            pltpu.SemaphoreType.DMA,
        ],
    )
    def deg_kernel(dst3_h, ones_h, z_h, out_h, di2, ones_v, acc, ssem):
        c = lax.axis_index("c")
        s = lax.axis_index("s")
        wid = c * _NS + s
        row0 = s * nps
        pltpu.sync_copy(ones_h, ones_v)
        pltpu.sync_copy(dst3_h.at[wid], di2)
        pltpu.sync_copy(z_h, acc.at[pl.ds(row0, nps)])
        plsc.subcore_barrier()

        def s_start(j):
            pltpu.async_copy(ones_v, acc.at[di2.at[j]], ssem, add=True)

        def s_wait():
            pltpu.make_async_copy(ones_v, acc.at[di2.at[0]], ssem).wait()

        for b in range(win):
            s_start(b)

        def body(j, carry):
            s_wait()
            s_start(j + win)
            return carry

        lax.fori_loop(0, nchunk - win, body, 0)
        for _ in range(win):
            s_wait()
        plsc.subcore_barrier()
        pltpu.sync_copy(acc.at[pl.ds(row0, nps)], out_h.at[c, pl.ds(row0, nps)])

    return deg_kernel(dst3, ones_rows, zrows)


_NB = 4


def _sc_segsum(emb, src4, dst4, zrows, npad, h, t0, t1):
    nps = npad // _NS
    assert t0 % 2 == 0 and t1 % 2 == 0 and min(t0, t1) >= 4
    mesh = plsc.VectorSubcoreMesh(core_axis_name="c", subcore_axis_name="s",
                                  num_cores=_NC, num_subcores=_NS)

    @functools.partial(
        pl.kernel,
        out_type=jax.ShapeDtypeStruct((_NC, npad, h), jnp.float32),
        mesh=mesh,
        scratch_types=[
            [pltpu.VMEM((_NB, _K), jnp.int32)] * 2,
            [pltpu.VMEM((_NB, _K), jnp.int32)] * 2,
            [pltpu.VMEM((_K, h), jnp.float32)] * _NB,
            pltpu.VMEM_SHARED((npad, h), jnp.float32),
            [pltpu.SemaphoreType.DMA] * 2,
            [pltpu.SemaphoreType.DMA] * _NB,
            [pltpu.SemaphoreType.DMA] * _NB,
        ],
    )
    def seg_kernel(emb_h, src4_h, dst4_h, z_h, out_h, is2, id2, rows, acc,
                   isem, gsem, ssem):
        c = lax.axis_index("c")
        s = lax.axis_index("s")
        row0 = s * nps
        base = jnp.where(c == 0, s * t0, _NS * t0 + s * t1)
        tg = jnp.where(c == 0, t0, t1)

        def i_start(g, p):
            pltpu.async_copy(src4_h.at[base + g], is2[p], isem[p])
            pltpu.async_copy(dst4_h.at[base + g], id2[p], isem[p])

        def i_wait(p):
            pltpu.make_async_copy(src4_h.at[base], is2[p], isem[p]).wait()
            pltpu.make_async_copy(src4_h.at[base], id2[p], isem[p]).wait()

        def g_start(p, b):
            pltpu.async_copy(emb_h.at[is2[p].at[b]], rows[b], gsem[b])

        def g_wait(p, b):
            pltpu.make_async_copy(emb_h.at[is2[p].at[b]], rows[b],
                                  gsem[b]).wait()

        def s_start(p, b):
            pltpu.async_copy(rows[b], acc.at[id2[p].at[b]], ssem[b], add=True)

        def s_wait(p, b):
            pltpu.make_async_copy(rows[b], acc.at[id2[p].at[b]],
                                  ssem[b]).wait()

        def process_group(g, p, pre):
            if pre:
                i_start(g + 1, 1 - p)
            for b in range(_NB):
                g_wait(p, b)
                s_start(p, b)
            if pre:
                i_wait(1 - p)
            for b in range(_NB):
                s_wait(p, b)
                if pre:
                    g_start(1 - p, b)

        i_start(0, 0)
        pltpu.sync_copy(z_h, acc.at[pl.ds(row0, nps)])
        plsc.subcore_barrier()
        i_wait(0)
        for b in range(_NB):
            g_start(0, b)

        def pair(t, carry):
            process_group(2 * t, 0, True)
            process_group(2 * t + 1, 1, True)
            return carry

        lax.fori_loop(0, tg // 2 - 1, pair, 0)
        process_group(tg - 2, 0, True)
        process_group(tg - 1, 1, False)

        plsc.subcore_barrier()
        pltpu.sync_copy(acc.at[pl.ds(row0, nps)], out_h.at[c, pl.ds(row0, nps)])

    return seg_kernel(emb, src4, dst4, zrows)


def kernel(feature, semantic, edge_index, W1s, b1s, W1m, b1m, W2, b2, alpha,
           beta, num_iterations):
    n, _ = feature.shape
    h = W2.shape[0]
    e = edge_index.shape[1]
    nw = _NC * _NS

    npad = -(-(n + 1) // (_NS * 8)) * (_NS * 8)

    gsz = _NB * _K
    spc = -(-e // (nw * gsz))
    if spc % 2:
        spc += 1
    t0 = 3 * spc // 2
    t1 = 2 * spc - t0
    nslab = _NS * (t0 + t1)
    e_pad = nslab * gsz
    src_p = jnp.concatenate(
        [edge_index[0], jnp.zeros((e_pad - e,), jnp.int32)])
    dst_p = jnp.concatenate(
        [edge_index[1], jnp.full((e_pad - e,), n, jnp.int32)])
    nchunk = (t0 + t1) * _NB // 2
    src4 = src_p.reshape(nslab, _NB, _K)
    dst4 = dst_p.reshape(nslab, _NB, _K)
    dst3 = dst_p.reshape(nw, nchunk, _K)

    alpha2 = jnp.asarray(alpha, jnp.float32).reshape(1, 1)
    beta2 = jnp.asarray(beta, jnp.float32).reshape(1, 1)
    b1s2 = b1s.reshape(1, -1)
    b1m2 = b1m.reshape(1, -1)
    b22 = b2.reshape(1, -1)

    nps = npad // _NS
    zrows = jnp.zeros((nps, h), jnp.float32)
    ones_rows = jnp.ones((_K, h), jnp.float32)

    degp = _sc_degree(dst3, ones_rows, zrows, npad, h, nchunk)
    dinv = _tc_dinv(degp, npad, h)
    emb0 = _tc_init(feature, semantic, W1s, b1s2, W1m, b1m2, alpha2, beta2,
                    blk=2000)

    def body(_, emb):
        parts = _sc_segsum(emb, src4, dst4, zrows, npad, h, t0, t1)
        return _tc_update(emb, parts, dinv, W2, b22, alpha2, beta2, blk=2000)

    return lax.fori_loop(0, num_iterations, body, emb0)


# speedup vs baseline: 1.2033x; 1.0276x over previous
import functools

import jax
import jax.numpy as jnp
from jax import lax
from jax.experimental import pallas as pl
from jax.experimental.pallas import tpu as pltpu
from jax.experimental.pallas import tpu_sc as plsc

_NC = 2
_NS = 16
_K = 80
_SLOPE = 0.01


def _lrelu(x):
    return jnp.where(x >= 0, x, _SLOPE * x)


def _tc_init(feature, semantic, w1s, b1s2, w1m, b1m2, alpha2, beta2, blk):
    n, f = feature.shape
    s_dim = semantic.shape[1]
    h = w1s.shape[0]

    def body(f_r, se_r, w1s_r, b1s_r, w1m_r, b1m_r, a_r, b_r, o_r):
        a = a_r[0, 0]
        b = b_r[0, 0]
        x = lax.dot_general(f_r[...], w1s_r[...], (((1,), (1,)), ((), ())),
                            preferred_element_type=jnp.float32) + b1s_r[...]
        y = lax.dot_general(se_r[...], w1m_r[...], (((1,), (1,)), ((), ())),
                            preferred_element_type=jnp.float32) + b1m_r[...]
        o_r[...] = a * _lrelu(x) + b * _lrelu(y)

    return pl.pallas_call(
        body,
        grid=(n // blk,),
        in_specs=[
            pl.BlockSpec((blk, f), lambda i: (i, 0)),
            pl.BlockSpec((blk, s_dim), lambda i: (i, 0)),
            pl.BlockSpec((h, f), lambda i: (0, 0)),
            pl.BlockSpec((1, h), lambda i: (0, 0)),
            pl.BlockSpec((h, s_dim), lambda i: (0, 0)),
            pl.BlockSpec((1, h), lambda i: (0, 0)),
            pl.BlockSpec((1, 1), lambda i: (0, 0)),
            pl.BlockSpec((1, 1), lambda i: (0, 0)),
        ],
        out_specs=pl.BlockSpec((blk, h), lambda i: (i, 0)),
        out_shape=jax.ShapeDtypeStruct((n, h), jnp.float32),
    )(feature, semantic, w1s, b1s2, w1m, b1m2, alpha2, beta2)


def _tc_dinv(degparts, npad, h):
    def body(d_r, o_r):
        dv = d_r[...]
        o_r[...] = 1.0 / jnp.maximum(dv[0] + dv[1], 1.0)

    return pl.pallas_call(
        body,
        out_shape=jax.ShapeDtypeStruct((npad, h), jnp.float32),
    )(degparts)


def _tc_update(emb, parts, dinv, w2, b22, alpha2, beta2, blk):
    n, h = emb.shape

    def body(e_r, p_r, d_r, w2_r, b2_r, a_r, b_r, o_r):
        a = a_r[0, 0]
        b = b_r[0, 0]
        pv = p_r[...]
        msum = pv[0] + pv[1]
        comb = a * e_r[...] + b * (msum * d_r[...])
        z = lax.dot_general(comb, w2_r[...], (((1,), (1,)), ((), ())),
                            preferred_element_type=jnp.float32) + b2_r[...]
        o_r[...] = _lrelu(z)

    return pl.pallas_call(
        body,
        grid=(n // blk,),
        in_specs=[
            pl.BlockSpec((blk, h), lambda i: (i, 0)),
            pl.BlockSpec((_NC, blk, h), lambda i: (0, i, 0)),
            pl.BlockSpec((blk, h), lambda i: (i, 0)),
            pl.BlockSpec((h, h), lambda i: (0, 0)),
            pl.BlockSpec((1, h), lambda i: (0, 0)),
            pl.BlockSpec((1, 1), lambda i: (0, 0)),
            pl.BlockSpec((1, 1), lambda i: (0, 0)),
        ],
        out_specs=pl.BlockSpec((blk, h), lambda i: (i, 0)),
        out_shape=jax.ShapeDtypeStruct((n, h), jnp.float32),
    )(emb, parts, dinv, w2, b22, alpha2, beta2)


def _sc_degree(dst3, ones_rows, zrows, npad, h, nchunk):
    nps = npad // _NS
    win = 8
    mesh = plsc.VectorSubcoreMesh(core_axis_name="c", subcore_axis_name="s",
                                  num_cores=_NC, num_subcores=_NS)

    @functools.partial(
        pl.kernel,
        out_type=jax.ShapeDtypeStruct((_NC, npad, h), jnp.float32),
        mesh=mesh,
        scratch_types=[
            pltpu.VMEM((nchunk, _K), jnp.int32),
            pltpu.VMEM((_K, h), jnp.float32),
            pltpu.VMEM_SHARED((npad, h), jnp.float32),
            pltpu.SemaphoreType.DMA,
        ],
    )
    def deg_kernel(dst3_h, ones_h, z_h, out_h, di2, ones_v, acc, ssem):
        c = lax.axis_index("c")
        s = lax.axis_index("s")
        wid = c * _NS + s
        row0 = s * nps
        pltpu.sync_copy(ones_h, ones_v)
        pltpu.sync_copy(dst3_h.at[wid], di2)
        pltpu.sync_copy(z_h, acc.at[pl.ds(row0, nps)])
        plsc.subcore_barrier()

        def s_start(j):
            pltpu.async_copy(ones_v, acc.at[di2.at[j]], ssem, add=True)

        def s_wait():
            pltpu.make_async_copy(ones_v, acc.at[di2.at[0]], ssem).wait()

        for b in range(win):
            s_start(b)

        def body(j, carry):
            s_wait()
            s_start(j + win)
            return carry

        lax.fori_loop(0, nchunk - win, body, 0)
        for _ in range(win):
            s_wait()
        plsc.subcore_barrier()
        pltpu.sync_copy(acc.at[pl.ds(row0, nps)], out_h.at[c, pl.ds(row0, nps)])

    return deg_kernel(dst3, ones_rows, zrows)


_NB = 4


def _sc_segsum(emb, src4, dst4, zrows, npad, h, t0, t1):
    nps = npad // _NS
    assert t0 % 2 == 0 and t1 % 2 == 0 and min(t0, t1) >= 4
    mesh = plsc.VectorSubcoreMesh(core_axis_name="c", subcore_axis_name="s",
                                  num_cores=_NC, num_subcores=_NS)

    @functools.partial(
        pl.kernel,
        out_type=jax.ShapeDtypeStruct((_NC, npad, h), jnp.float32),
        mesh=mesh,
        scratch_types=[
            [pltpu.VMEM((_NB, _K), jnp.int32)] * 2,
            [pltpu.VMEM((_NB, _K), jnp.int32)] * 2,
            [pltpu.VMEM((_K, h), jnp.float32)] * _NB,
            pltpu.VMEM_SHARED((npad, h), jnp.float32),
            [pltpu.SemaphoreType.DMA] * 2,
            [pltpu.SemaphoreType.DMA] * _NB,
            [pltpu.SemaphoreType.DMA] * _NB,
        ],
    )
    def seg_kernel(emb_h, src4_h, dst4_h, z_h, out_h, is2, id2, rows, acc,
                   isem, gsem, ssem):
        c = lax.axis_index("c")
        s = lax.axis_index("s")
        row0 = s * nps
        base = jnp.where(c == 0, s * t0, _NS * t0 + s * t1)
        tg = jnp.where(c == 0, t0, t1)

        def i_start(g, p):
            pltpu.async_copy(src4_h.at[base + g], is2[p], isem[p])
            pltpu.async_copy(dst4_h.at[base + g], id2[p], isem[p])

        def i_wait(p):
            pltpu.make_async_copy(src4_h.at[base], is2[p], isem[p]).wait()
            pltpu.make_async_copy(src4_h.at[base], id2[p], isem[p]).wait()

        def g_start(p, b):
            pltpu.async_copy(emb_h.at[is2[p].at[b]], rows[b], gsem[b])

        def g_wait(p, b):
            pltpu.make_async_copy(emb_h.at[is2[p].at[b]], rows[b],
                                  gsem[b]).wait()

        def s_start(p, b):
            pltpu.async_copy(rows[b], acc.at[id2[p].at[b]], ssem[b], add=True)

        def s_wait(p, b):
            pltpu.make_async_copy(rows[b], acc.at[id2[p].at[b]],
                                  ssem[b]).wait()

        def process_group(g, p, pre):
            if pre:
                i_start(g + 1, 1 - p)
            for b in range(_NB):
                g_wait(p, b)
                s_start(p, b)
            if pre:
                i_wait(1 - p)
            for b in range(_NB):
                s_wait(p, b)
                if pre:
                    g_start(1 - p, b)

        i_start(0, 0)
        pltpu.sync_copy(z_h, acc.at[pl.ds(row0, nps)])
        plsc.subcore_barrier()
        i_wait(0)
        for b in range(_NB):
            g_start(0, b)

        def pair(t, carry):
            process_group(2 * t, 0, True)
            process_group(2 * t + 1, 1, True)
            return carry

        lax.fori_loop(0, tg // 2 - 1, pair, 0)
        process_group(tg - 2, 0, True)
        process_group(tg - 1, 1, False)

        plsc.subcore_barrier()
        pltpu.sync_copy(acc.at[pl.ds(row0, nps)], out_h.at[c, pl.ds(row0, nps)])

    return seg_kernel(emb, src4, dst4, zrows)


def kernel(feature, semantic, edge_index, W1s, b1s, W1m, b1m, W2, b2, alpha,
           beta, num_iterations):
    n, _ = feature.shape
    h = W2.shape[0]
    e = edge_index.shape[1]
    nw = _NC * _NS

    npad = -(-(n + 1) // (_NS * 8)) * (_NS * 8)

    gsz = _NB * _K
    spc = -(-e // (nw * gsz))
    if spc % 2:
        spc += 1
    t1 = 4
    t0 = 2 * spc - t1
    nslab = _NS * (t0 + t1)
    e_pad = nslab * gsz
    src_p = jnp.concatenate(
        [edge_index[0], jnp.zeros((e_pad - e,), jnp.int32)])
    dst_p = jnp.concatenate(
        [edge_index[1], jnp.full((e_pad - e,), n, jnp.int32)])
    nchunk = (t0 + t1) * _NB // 2
    src4 = src_p.reshape(nslab, _NB, _K)
    dst4 = dst_p.reshape(nslab, _NB, _K)
    dst3 = dst_p.reshape(nw, nchunk, _K)

    alpha2 = jnp.asarray(alpha, jnp.float32).reshape(1, 1)
    beta2 = jnp.asarray(beta, jnp.float32).reshape(1, 1)
    b1s2 = b1s.reshape(1, -1)
    b1m2 = b1m.reshape(1, -1)
    b22 = b2.reshape(1, -1)

    nps = npad // _NS
    zrows = jnp.zeros((nps, h), jnp.float32)
    ones_rows = jnp.ones((_K, h), jnp.float32)

    degp = _sc_degree(dst3, ones_rows, zrows, npad, h, nchunk)
    dinv = _tc_dinv(degp, npad, h)
    emb0 = _tc_init(feature, semantic, W1s, b1s2, W1m, b1m2, alpha2, beta2,
                    blk=2000)

    def body(_, emb):
        parts = _sc_segsum(emb, src4, dst4, zrows, npad, h, t0, t1)
        return _tc_update(emb, parts, dinv, W2, b22, alpha2, beta2, blk=2000)

    return lax.fori_loop(0, num_iterations, body, emb0)
